# Initial kernel scaffold; baseline (speedup 1.0000x reference)
import jax
import jax.numpy as jnp
from jax.experimental import pallas as pl


def kernel(xyz, new_xyz):
    raise NotImplementedError("write your pallas kernel here")


# speedup vs baseline: 7.4969x; 7.4969x over previous
import functools

import jax
import jax.numpy as jnp
from jax.experimental import pallas as pl

NSAMPLE = 32


def _knn_kernel(xyz_ref, new_xyz_ref, out_ref):
    p = xyz_ref[0]
    q = new_xyz_ref[0]
    n = p.shape[0]
    qb = q.shape[0]

    qr = q.astype(jnp.bfloat16).astype(jnp.float32)
    pr = p.astype(jnp.bfloat16).astype(jnp.float32)
    dots = (qr[:, 0:1] * pr[:, 0][None, :]
            + qr[:, 1:2] * pr[:, 1][None, :]
            + qr[:, 2:3] * pr[:, 2][None, :])
    qn = q[:, 0] * q[:, 0] + q[:, 1] * q[:, 1] + q[:, 2] * q[:, 2]
    pn = p[:, 0] * p[:, 0] + p[:, 1] * p[:, 1] + p[:, 2] * p[:, 2]
    dist = -2.0 * dots
    dist = dist + qn[:, None]
    dist = dist + pn[None, :]

    lane_iota = jax.lax.broadcasted_iota(jnp.int32, (qb, n), 1)
    inf = jnp.float32(jnp.inf)
    big = jnp.int32(n)

    cols = []
    for _ in range(NSAMPLE):
        m = jnp.min(dist, axis=1)
        cand = jnp.where(dist == m[:, None], lane_iota, big)
        idx = jnp.min(cand, axis=1)
        cols.append(idx)
        dist = jnp.where(lane_iota == idx[:, None], inf, dist)
    out = jnp.stack(cols, axis=1)
    out_ref[0] = out


@jax.jit
def kernel(xyz, new_xyz):
    b, n, _ = xyz.shape
    _, s, _ = new_xyz.shape
    qb = min(s, 128)
    grid = (b, s // qb)
    return pl.pallas_call(
        _knn_kernel,
        grid=grid,
        in_specs=[
            pl.BlockSpec((1, n, 3), lambda i, j: (i, 0, 0)),
            pl.BlockSpec((1, qb, 3), lambda i, j: (i, j, 0)),
        ],
        out_specs=pl.BlockSpec((1, qb, NSAMPLE), lambda i, j: (i, j, 0)),
        out_shape=jax.ShapeDtypeStruct((b, s, NSAMPLE), jnp.int32),
    )(xyz, new_xyz)
